# baseline (speedup 1.0000x reference)
import functools

import jax
import jax.numpy as jnp
from jax import lax
from jax.experimental import pallas as pl
from jax.experimental.pallas import tpu as pltpu
from jax.experimental.pallas import tpu_sc as plsc

VOCAB_LOCAL = 100000
N_EMBD = 1024
NUM_TOKENS = 8192

NUM_CORES = 2
NUM_SUBCORES = 16
LANES = 16
NUM_WORKERS = NUM_CORES * NUM_SUBCORES
TOKENS_PER_WORKER = NUM_TOKENS // NUM_WORKERS
CHUNK = 16
HALF = CHUNK // 2
NUM_CHUNKS = TOKENS_PER_WORKER // CHUNK
NBUF = 4


@functools.partial(
    pl.kernel,
    mesh=plsc.VectorSubcoreMesh(core_axis_name="c", subcore_axis_name="s"),
    out_type=jax.ShapeDtypeStruct((NUM_TOKENS, N_EMBD), jnp.float32),
    scratch_types=[
        pltpu.VMEM((TOKENS_PER_WORKER,), jnp.int32),
        pltpu.VMEM((CHUNK, N_EMBD), jnp.float32),
        pltpu.VMEM((CHUNK, N_EMBD), jnp.float32),
        pltpu.VMEM((CHUNK, N_EMBD), jnp.float32),
        pltpu.VMEM((CHUNK, N_EMBD), jnp.float32),
        pltpu.SemaphoreType.DMA,
        pltpu.SemaphoreType.DMA,
        pltpu.SemaphoreType.DMA,
        pltpu.SemaphoreType.DMA,
        pltpu.SemaphoreType.DMA,
        pltpu.SemaphoreType.DMA,
        pltpu.SemaphoreType.DMA,
        pltpu.SemaphoreType.DMA,
        pltpu.SemaphoreType.DMA,
    ],
)
def _emb_double(table_hbm, x_hbm, out_hbm, idx_v, b0, b1, b2, b3,
                g0, g1, g2, g3, s0, s1, s2, s3, isem):
    bufs = (b0, b1, b2, b3)
    gsems = (g0, g1, g2, g3)
    ssems = (s0, s1, s2, s3)

    wid = lax.axis_index("s") * NUM_CORES + lax.axis_index("c")
    row_base = wid * TOKENS_PER_WORKER

    def gather_copy(k, b):
        return pltpu.make_async_copy(
            table_hbm.at[idx_v.at[pl.ds(k * CHUNK, CHUNK)]], bufs[b], gsems[b])

    def chunk_scatter_wait(b):
        pltpu.make_async_copy(
            bufs[b], out_hbm.at[pl.ds(row_base, CHUNK)], ssems[b]).wait()

    def start_scatter_half(k, b, h):
        return pltpu.async_copy(
            bufs[b].at[pl.ds(h * HALF, HALF)],
            out_hbm.at[pl.ds(row_base + k * CHUNK + h * HALF, HALF)],
            ssems[b])

    def double_half(b, h):
        buf = bufs[b]
        eighth_cols = N_EMBD // 8

        def group_body(g, carry):
            r = g >> 3
            cbase = (g & 7) * eighth_cols
            for j in range(eighth_cols // LANES):
                v = buf[r, pl.ds(cbase + j * LANES, LANES)]
                buf[r, pl.ds(cbase + j * LANES, LANES)] = v + v
            return carry

        lax.fori_loop(8 * h * HALF, 8 * (h + 1) * HALF, group_body, 0)

    pltpu.sync_copy(x_hbm.at[pl.ds(row_base, CHUNK)],
                    idx_v.at[pl.ds(0, CHUNK)])
    gather_copy(0, 0).start()
    pltpu.async_copy(
        x_hbm.at[pl.ds(row_base + CHUNK, TOKENS_PER_WORKER - CHUNK)],
        idx_v.at[pl.ds(CHUNK, TOKENS_PER_WORKER - CHUNK)], isem).wait()
    for b in range(1, NBUF - 1):
        gather_copy(b, b).start()

    def outer(i, carry):
        kbase = i * NBUF
        for b in range(NBUF):
            k = kbase + b
            bp = (b - 1) % NBUF

            @pl.when(k + NBUF - 1 < NUM_CHUNKS)
            def _():
                @pl.when(k >= 1)
                def _():
                    chunk_scatter_wait(bp)
                gather_copy(k + NBUF - 1, bp).start()

            gather_copy(k, b).wait()
            for h in range(CHUNK // HALF):
                double_half(b, h)
                start_scatter_half(k, b, h)

        return carry

    lax.fori_loop(0, NUM_CHUNKS // NBUF, outer, 0)

    for b in range(NBUF):
        chunk_scatter_wait(b)


def kernel(x, emb_weight):
    return _emb_double(emb_weight, x.astype(jnp.int32))


# speedup vs baseline: 1.0004x; 1.0004x over previous
import functools

import jax
import jax.numpy as jnp
from jax import lax
from jax.experimental import pallas as pl
from jax.experimental.pallas import tpu as pltpu
from jax.experimental.pallas import tpu_sc as plsc

VOCAB_LOCAL = 100000
N_EMBD = 1024
NUM_TOKENS = 8192

NUM_CORES = 2
NUM_SUBCORES = 16
LANES = 16
NUM_WORKERS = NUM_CORES * NUM_SUBCORES
TOKENS_PER_WORKER = NUM_TOKENS // NUM_WORKERS
CHUNK = 16
HALF = CHUNK // 2
NUM_CHUNKS = TOKENS_PER_WORKER // CHUNK
NBUF = 4


@functools.partial(
    pl.kernel,
    mesh=plsc.VectorSubcoreMesh(core_axis_name="c", subcore_axis_name="s"),
    out_type=jax.ShapeDtypeStruct((NUM_TOKENS, N_EMBD), jnp.float32),
    scratch_types=[
        pltpu.VMEM((TOKENS_PER_WORKER,), jnp.int32),
        pltpu.VMEM((CHUNK, N_EMBD), jnp.float32),
        pltpu.VMEM((CHUNK, N_EMBD), jnp.float32),
        pltpu.VMEM((CHUNK, N_EMBD), jnp.float32),
        pltpu.VMEM((CHUNK, N_EMBD), jnp.float32),
        pltpu.SemaphoreType.DMA,
        pltpu.SemaphoreType.DMA,
        pltpu.SemaphoreType.DMA,
        pltpu.SemaphoreType.DMA,
        pltpu.SemaphoreType.DMA,
        pltpu.SemaphoreType.DMA,
        pltpu.SemaphoreType.DMA,
        pltpu.SemaphoreType.DMA,
        pltpu.SemaphoreType.DMA,
    ],
)
def _emb_double(table_hbm, x_hbm, out_hbm, idx_v, b0, b1, b2, b3,
                g0, g1, g2, g3, s0, s1, s2, s3, isem):
    bufs = (b0, b1, b2, b3)
    gsems = (g0, g1, g2, g3)
    ssems = (s0, s1, s2, s3)

    wid = lax.axis_index("s") * NUM_CORES + lax.axis_index("c")
    row_base = wid * TOKENS_PER_WORKER

    def gather_copy(k, b):
        return pltpu.make_async_copy(
            table_hbm.at[idx_v.at[pl.ds(k * CHUNK, CHUNK)]], bufs[b], gsems[b])

    def chunk_scatter_wait(b):
        pltpu.make_async_copy(
            bufs[b], out_hbm.at[pl.ds(row_base, CHUNK)], ssems[b]).wait()

    def start_scatter_half(k, b, h):
        return pltpu.async_copy(
            bufs[b].at[pl.ds(h * HALF, HALF)],
            out_hbm.at[pl.ds(row_base + k * CHUNK + h * HALF, HALF)],
            ssems[b])

    def double_half(b, h):
        buf = bufs[b]
        eighth_cols = N_EMBD // 8

        def group_body(g, carry):
            r = g >> 3
            cbase = (g & 7) * eighth_cols
            for j in range(eighth_cols // LANES):
                v = buf[r, pl.ds(cbase + j * LANES, LANES)]
                buf[r, pl.ds(cbase + j * LANES, LANES)] = v + v
            return carry

        lax.fori_loop(8 * h * HALF, 8 * (h + 1) * HALF, group_body, 0)

    pltpu.sync_copy(x_hbm.at[pl.ds(row_base, CHUNK)],
                    idx_v.at[pl.ds(0, CHUNK)])
    gather_copy(0, 0).start()
    pltpu.async_copy(
        x_hbm.at[pl.ds(row_base + CHUNK, TOKENS_PER_WORKER - CHUNK)],
        idx_v.at[pl.ds(CHUNK, TOKENS_PER_WORKER - CHUNK)], isem).wait()
    for b in range(1, NBUF - 1):
        gather_copy(b, b).start()

    def outer(i, carry):
        kbase = i * NBUF
        for b in range(NBUF):
            k = kbase + b
            bp = (b - 1) % NBUF

            @pl.when(k + NBUF - 1 < NUM_CHUNKS)
            def _():
                @pl.when(k >= 1)
                def _():
                    chunk_scatter_wait(bp)
                gather_copy(k + NBUF - 1, bp).start()

            gather_copy(k, b).wait()
            double_half(b, 0)
            double_half(b, 1)
            pltpu.async_copy(
                bufs[b], out_hbm.at[pl.ds(row_base + k * CHUNK, CHUNK)],
                ssems[b])

        return carry

    lax.fori_loop(0, NUM_CHUNKS // NBUF, outer, 0)

    for b in range(NBUF):
        chunk_scatter_wait(b)


def kernel(x, emb_weight):
    return _emb_double(emb_weight, x.astype(jnp.int32))


# speedup vs baseline: 1.0057x; 1.0053x over previous
import functools

import jax
import jax.numpy as jnp
from jax import lax
from jax.experimental import pallas as pl
from jax.experimental.pallas import tpu as pltpu
from jax.experimental.pallas import tpu_sc as plsc

VOCAB_LOCAL = 100000
N_EMBD = 1024
NUM_TOKENS = 8192

NUM_CORES = 2
NUM_SUBCORES = 16
LANES = 16
NUM_WORKERS = NUM_CORES * NUM_SUBCORES
TOKENS_PER_WORKER = NUM_TOKENS // NUM_WORKERS
CHUNK = 16
HALF = CHUNK // 2
NUM_CHUNKS = TOKENS_PER_WORKER // CHUNK
NBUF = 4


@functools.partial(
    pl.kernel,
    mesh=plsc.VectorSubcoreMesh(core_axis_name="c", subcore_axis_name="s"),
    out_type=jax.ShapeDtypeStruct((NUM_TOKENS, N_EMBD), jnp.float32),
    scratch_types=[
        pltpu.VMEM((TOKENS_PER_WORKER,), jnp.int32),
        pltpu.VMEM((CHUNK, N_EMBD), jnp.float32),
        pltpu.VMEM((CHUNK, N_EMBD), jnp.float32),
        pltpu.VMEM((CHUNK, N_EMBD), jnp.float32),
        pltpu.VMEM((CHUNK, N_EMBD), jnp.float32),
        pltpu.SemaphoreType.DMA,
        pltpu.SemaphoreType.DMA,
        pltpu.SemaphoreType.DMA,
        pltpu.SemaphoreType.DMA,
        pltpu.SemaphoreType.DMA,
        pltpu.SemaphoreType.DMA,
        pltpu.SemaphoreType.DMA,
        pltpu.SemaphoreType.DMA,
        pltpu.SemaphoreType.DMA,
    ],
)
def _emb_double(table_hbm, x_hbm, out_hbm, idx_v, b0, b1, b2, b3,
                g0, g1, g2, g3, s0, s1, s2, s3, isem):
    bufs = (b0, b1, b2, b3)
    gsems = (g0, g1, g2, g3)
    ssems = (s0, s1, s2, s3)

    wid = lax.axis_index("s") * NUM_CORES + lax.axis_index("c")
    row_base = wid * TOKENS_PER_WORKER

    def gather_copy(k, b):
        return pltpu.make_async_copy(
            table_hbm.at[idx_v.at[pl.ds(k * CHUNK, CHUNK)]], bufs[b], gsems[b])

    def chunk_scatter_wait(b):
        pltpu.make_async_copy(
            bufs[b], out_hbm.at[pl.ds(row_base, CHUNK)], ssems[b]).wait()

    def start_scatter_half(k, b, h):
        return pltpu.async_copy(
            bufs[b].at[pl.ds(h * HALF, HALF)],
            out_hbm.at[pl.ds(row_base + k * CHUNK + h * HALF, HALF)],
            ssems[b])

    def double_half(b, h):
        buf = bufs[b]
        eighth_cols = N_EMBD // 8

        def group_body(g, carry):
            r = g >> 3
            cbase = (g & 7) * eighth_cols
            for j in range(eighth_cols // LANES):
                v = buf[r, pl.ds(cbase + j * LANES, LANES)]
                buf[r, pl.ds(cbase + j * LANES, LANES)] = v + v
            return carry

        lax.fori_loop(8 * h * HALF, 8 * (h + 1) * HALF, group_body, 0)

    pltpu.sync_copy(x_hbm.at[pl.ds(row_base, CHUNK)],
                    idx_v.at[pl.ds(0, CHUNK)])
    gather_copy(0, 0).start()
    pltpu.async_copy(
        x_hbm.at[pl.ds(row_base + CHUNK, TOKENS_PER_WORKER - CHUNK)],
        idx_v.at[pl.ds(CHUNK, TOKENS_PER_WORKER - CHUNK)], isem).wait()
    for b in range(1, NBUF - 1):
        gather_copy(b, b).start()

    def outer(i, carry):
        kbase = i * NBUF
        for b in range(NBUF):
            k = kbase + b
            bp = (b - 1) % NBUF

            @pl.when(k + NBUF - 1 < NUM_CHUNKS)
            def _():
                @pl.when(k >= 1)
                def _():
                    chunk_scatter_wait(bp)
                gather_copy(k + NBUF - 1, bp).start()

            gather_copy(k, b).wait()
            for h in range(CHUNK // HALF):
                double_half(b, h)
                start_scatter_half(k, b, h)

        return carry

    lax.fori_loop(0, NUM_CHUNKS // NBUF, outer, 0)

    for b in range(NBUF):
        chunk_scatter_wait(b)


def kernel(x, emb_weight):
    return _emb_double(emb_weight, x.astype(jnp.int32))
